# Initial kernel scaffold; baseline (speedup 1.0000x reference)
import jax
import jax.numpy as jnp
from jax.experimental import pallas as pl


def kernel(node_feats, edge_feats, edge_index, W1a, b1a, W1b, b1b, W2a, b2a, W2b, b2b):
    raise NotImplementedError("write your pallas kernel here")


# speedup vs baseline: 2.3797x; 2.3797x over previous
import functools

import jax
import jax.numpy as jnp
from jax import lax
from jax.experimental import pallas as pl
from jax.experimental.pallas import tpu as pltpu
from jax.experimental.pallas import tpu_sc as plsc

NC = 2
NS = 16
NW = NC * NS
LANES = 16
CHUNK = 80


def _mm_bias(x, w, b, block_rows):
    rows, k = x.shape
    dout = w.shape[1]

    def body(x_ref, w_ref, b_ref, o_ref):
        o_ref[...] = (
            jnp.dot(x_ref[...], w_ref[...], preferred_element_type=jnp.float32)
            + b_ref[...]
        )

    return pl.pallas_call(
        body,
        grid=(rows // block_rows,),
        in_specs=[
            pl.BlockSpec((block_rows, k), lambda i: (i, 0)),
            pl.BlockSpec((k, dout), lambda i: (0, 0)),
            pl.BlockSpec((1, dout), lambda i: (0, 0)),
        ],
        out_specs=pl.BlockSpec((block_rows, dout), lambda i: (i, 0)),
        out_shape=jax.ShapeDtypeStruct((rows, dout), jnp.float32),
    )(x, w, b)


def _sc_gather_relu_scatter(ewb, g, src3, dst3, n_pad):
    e_total, d = ewb.shape
    nch = e_total // (NW * CHUNK)
    zr = n_pad // NS
    mesh = plsc.VectorSubcoreMesh(core_axis_name="c", subcore_axis_name="s")

    @functools.partial(
        pl.kernel,
        out_type=jax.ShapeDtypeStruct((NC, n_pad, d), jnp.float32),
        mesh=mesh,
        scratch_types=[
            pltpu.VMEM_SHARED((n_pad, d), jnp.float32),
            pltpu.VMEM((CHUNK,), jnp.int32),
            pltpu.VMEM((CHUNK,), jnp.int32),
            pltpu.VMEM((CHUNK, d), jnp.float32),
            pltpu.VMEM((CHUNK, d), jnp.float32),
            pltpu.SemaphoreType.DMA,
        ],
    )
    def k(ewb_hbm, g_hbm, src_hbm, dst_hbm, s_out,
          s_sh, src_v, dst_v, ewb_v, grow_v, sem):
        c = lax.axis_index("c")
        s = lax.axis_index("s")
        wid = s * NC + c
        z16 = jnp.zeros((LANES,), jnp.float32)

        def zrow(r, carry):
            for cc in range(d // LANES):
                grow_v[r, pl.ds(cc * LANES, LANES)] = z16
            return carry

        lax.fori_loop(0, CHUNK, zrow, 0)
        for kk in range(zr // CHUNK):
            pltpu.sync_copy(grow_v, s_sh.at[pl.ds(s * zr + kk * CHUNK, CHUNK)])
        plsc.subcore_barrier()

        def chunk_body(j, carry):
            pltpu.sync_copy(src_hbm.at[wid, j], src_v)
            pltpu.sync_copy(dst_hbm.at[wid, j], dst_v)
            base = (wid * nch + j) * CHUNK
            pltpu.sync_copy(ewb_hbm.at[pl.ds(base, CHUNK)], ewb_v)
            pltpu.async_copy(g_hbm.at[src_v], grow_v, sem).wait()

            def vrow(r, rcarry):
                for cc in range(d // LANES):
                    sl = pl.ds(cc * LANES, LANES)
                    grow_v[r, sl] = jnp.maximum(grow_v[r, sl] + ewb_v[r, sl], 0.0)
                return rcarry

            lax.fori_loop(0, CHUNK, vrow, 0)
            pltpu.sync_copy(grow_v, s_sh.at[dst_v], add=True)
            return carry

        lax.fori_loop(0, nch, chunk_body, 0)
        plsc.subcore_barrier()
        for kk in range(zr // CHUNK):
            off = s * zr + kk * CHUNK
            pltpu.sync_copy(s_sh.at[pl.ds(off, CHUNK)], grow_v)
            pltpu.sync_copy(grow_v, s_out.at[c, pl.ds(off, CHUNK)])

    return k(ewb, g, src3, dst3)


def _sc_degree(dst3, n_pad, d, e_total):
    nch = e_total // (NW * CHUNK)
    zr = n_pad // NS
    mesh = plsc.VectorSubcoreMesh(core_axis_name="c", subcore_axis_name="s")

    @functools.partial(
        pl.kernel,
        out_type=jax.ShapeDtypeStruct((NC, n_pad, d), jnp.float32),
        mesh=mesh,
        scratch_types=[
            pltpu.VMEM_SHARED((n_pad, d), jnp.float32),
            pltpu.VMEM((CHUNK,), jnp.int32),
            pltpu.VMEM((CHUNK, d), jnp.float32),
        ],
    )
    def k(dst_hbm, deg_out, deg_sh, dst_v, ones_v):
        c = lax.axis_index("c")
        s = lax.axis_index("s")
        wid = s * NC + c
        z16 = jnp.zeros((LANES,), jnp.float32)
        o16 = jnp.full((LANES,), 1.0, jnp.float32)

        def zrow(r, carry):
            for cc in range(d // LANES):
                ones_v[r, pl.ds(cc * LANES, LANES)] = z16
            return carry

        lax.fori_loop(0, CHUNK, zrow, 0)
        for kk in range(zr // CHUNK):
            pltpu.sync_copy(ones_v, deg_sh.at[pl.ds(s * zr + kk * CHUNK, CHUNK)])

        def orow(r, carry):
            for cc in range(d // LANES):
                ones_v[r, pl.ds(cc * LANES, LANES)] = o16
            return carry

        lax.fori_loop(0, CHUNK, orow, 0)
        plsc.subcore_barrier()

        def chunk_body(j, carry):
            pltpu.sync_copy(dst_hbm.at[wid, j], dst_v)
            pltpu.sync_copy(ones_v, deg_sh.at[dst_v], add=True)
            return carry

        lax.fori_loop(0, nch, chunk_body, 0)
        plsc.subcore_barrier()
        for kk in range(zr // CHUNK):
            off = s * zr + kk * CHUNK
            pltpu.sync_copy(deg_sh.at[pl.ds(off, CHUNK)], ones_v)
            pltpu.sync_copy(ones_v, deg_out.at[c, pl.ds(off, CHUNK)])

    return k(dst3)


def _final_mlp(s_parts, deg_parts, node, w1b, b1b, w2a, b2a, w2b, b2b, block_rows):
    n, d = node.shape

    def body(sp, dp, nd, w1, v1, w2, v2, w3, v3, o):
        seg = sp[0] + sp[1]
        deg = dp[0, :, 0:1] + dp[1, :, 0:1]
        agg = jnp.dot(seg, w1[...], preferred_element_type=jnp.float32) + deg * v1[...]
        h = agg + nd[...]
        hid = jnp.maximum(
            jnp.dot(h, w2[...], preferred_element_type=jnp.float32) + v2[...], 0.0)
        o[...] = jnp.dot(hid, w3[...], preferred_element_type=jnp.float32) + v3[...]

    wspec = pl.BlockSpec((d, d), lambda i: (0, 0))
    bspec = pl.BlockSpec((1, d), lambda i: (0, 0))
    return pl.pallas_call(
        body,
        grid=(n // block_rows,),
        in_specs=[
            pl.BlockSpec((NC, block_rows, d), lambda i: (0, i, 0)),
            pl.BlockSpec((NC, block_rows, d), lambda i: (0, i, 0)),
            pl.BlockSpec((block_rows, d), lambda i: (i, 0)),
            wspec, bspec, wspec, bspec, wspec, bspec,
        ],
        out_specs=pl.BlockSpec((block_rows, d), lambda i: (i, 0)),
        out_shape=jax.ShapeDtypeStruct((n, d), jnp.float32),
    )(s_parts, deg_parts, node, w1b, b1b, w2a, b2a, w2b, b2b)


def kernel(node_feats, edge_feats, edge_index, W1a, b1a, W1b, b1b, W2a, b2a, W2b, b2b):
    n, d = node_feats.shape
    e = edge_feats.shape[0]
    src = edge_index[0].astype(jnp.int32)
    dst = edge_index[1].astype(jnp.int32)
    nch = e // (NW * CHUNK)
    src3 = src.reshape(NW, nch, CHUNK)
    dst3 = dst.reshape(NW, nch, CHUNK)
    tile_rows = NS * CHUNK
    n_pad = ((n + tile_rows - 1) // tile_rows) * tile_rows

    g = _mm_bias(node_feats, W1a[:d], jnp.zeros((1, d), jnp.float32), 2000)
    ewb = _mm_bias(edge_feats, W1a[d:], b1a.reshape(1, d), 2560)
    s_parts = _sc_gather_relu_scatter(ewb, g, src3, dst3, n_pad)
    deg_parts = _sc_degree(dst3, n_pad, d, e)
    return _final_mlp(s_parts, deg_parts, node_feats,
                      W1b, b1b.reshape(1, d), W2a, b2a.reshape(1, d),
                      W2b, b2b.reshape(1, d), 2000)


# speedup vs baseline: 3.5553x; 1.4940x over previous
import functools

import jax
import jax.numpy as jnp
from jax import lax
from jax.experimental import pallas as pl
from jax.experimental.pallas import tpu as pltpu
from jax.experimental.pallas import tpu_sc as plsc

NC = 2
NS = 16
NW = NC * NS
LANES = 16
CHUNK = 80


def _mm_bias(x, w, b, block_rows):
    rows, k = x.shape
    dout = w.shape[1]

    def body(x_ref, w_ref, b_ref, o_ref):
        o_ref[...] = (
            jnp.dot(x_ref[...], w_ref[...], preferred_element_type=jnp.float32)
            + b_ref[...]
        )

    return pl.pallas_call(
        body,
        grid=(rows // block_rows,),
        in_specs=[
            pl.BlockSpec((block_rows, k), lambda i: (i, 0)),
            pl.BlockSpec((k, dout), lambda i: (0, 0)),
            pl.BlockSpec((1, dout), lambda i: (0, 0)),
        ],
        out_specs=pl.BlockSpec((block_rows, dout), lambda i: (i, 0)),
        out_shape=jax.ShapeDtypeStruct((rows, dout), jnp.float32),
    )(x, w, b)


def _sc_gather_relu_scatter(ewb, g, src3, dst3, n_pad):
    e_total, d = ewb.shape
    nch = e_total // (NW * CHUNK)
    zr = n_pad // NS
    mesh = plsc.VectorSubcoreMesh(core_axis_name="c", subcore_axis_name="s")

    assert nch % 2 == 1 and nch >= 3

    @functools.partial(
        pl.kernel,
        out_type=jax.ShapeDtypeStruct((NC, n_pad, d), jnp.float32),
        mesh=mesh,
        scratch_types=[
            pltpu.VMEM_SHARED((n_pad, d), jnp.float32),
            pltpu.VMEM((CHUNK,), jnp.int32),
            pltpu.VMEM((CHUNK,), jnp.int32),
            pltpu.VMEM((CHUNK,), jnp.int32),
            pltpu.VMEM((CHUNK,), jnp.int32),
            pltpu.VMEM((CHUNK, d), jnp.float32),
            pltpu.VMEM((CHUNK, d), jnp.float32),
            pltpu.VMEM((CHUNK, d), jnp.float32),
            pltpu.VMEM((CHUNK, d), jnp.float32),
            pltpu.SemaphoreType.DMA,
            pltpu.SemaphoreType.DMA,
            pltpu.SemaphoreType.DMA,
            pltpu.SemaphoreType.DMA,
        ],
    )
    def k(ewb_hbm, g_hbm, src_hbm, dst_hbm, s_out,
          s_sh, src0, dst0, src1, dst1, ewb0, ewb1, grow0, grow1,
          sin0, sin1, gs0, gs1):
        c = lax.axis_index("c")
        s = lax.axis_index("s")
        wid = s * NC + c
        z16 = jnp.zeros((LANES,), jnp.float32)
        slot0 = (src0, dst0, ewb0, grow0, sin0, gs0)
        slot1 = (src1, dst1, ewb1, grow1, sin1, gs1)

        def issue_in(j, sl):
            src_b, dst_b, ewb_b, _, sem, _ = sl
            pltpu.async_copy(src_hbm.at[wid, j], src_b, sem)
            pltpu.async_copy(dst_hbm.at[wid, j], dst_b, sem)
            base = (wid * nch + j) * CHUNK
            pltpu.async_copy(ewb_hbm.at[pl.ds(base, CHUNK)], ewb_b, sem)

        def wait_in(sl):
            src_b, dst_b, ewb_b, _, sem, _ = sl
            pltpu.make_async_copy(src_hbm.at[wid, 0], src_b, sem).wait()
            pltpu.make_async_copy(dst_hbm.at[wid, 0], dst_b, sem).wait()
            pltpu.make_async_copy(ewb_hbm.at[pl.ds(0, CHUNK)], ewb_b, sem).wait()

        def issue_gather(sl):
            src_b, _, _, grow_b, _, gsem = sl
            pltpu.async_copy(g_hbm.at[src_b], grow_b, gsem)

        def wait_gather(sl):
            src_b, _, _, grow_b, _, gsem = sl
            pltpu.make_async_copy(g_hbm.at[src_b], grow_b, gsem).wait()

        def compute_scatter(sl):
            _, dst_b, ewb_b, grow_b, _, _ = sl

            def vrow(r, rcarry):
                for cc in range(d // LANES):
                    vs = pl.ds(cc * LANES, LANES)
                    grow_b[r, vs] = jnp.maximum(grow_b[r, vs] + ewb_b[r, vs], 0.0)
                return rcarry

            lax.fori_loop(0, CHUNK, vrow, 0)
            pltpu.sync_copy(grow_b, s_sh.at[dst_b], add=True)

        def zrow(r, carry):
            for cc in range(d // LANES):
                grow0[r, pl.ds(cc * LANES, LANES)] = z16
            return carry

        lax.fori_loop(0, CHUNK, zrow, 0)
        for kk in range(zr // CHUNK):
            pltpu.sync_copy(grow0, s_sh.at[pl.ds(s * zr + kk * CHUNK, CHUNK)])
        plsc.subcore_barrier()

        issue_in(0, slot0)
        issue_in(1, slot1)
        wait_in(slot0)
        issue_gather(slot0)

        def pairbody(kk, carry):
            j0 = 2 * kk
            wait_in(slot1)
            issue_gather(slot1)
            wait_gather(slot0)
            compute_scatter(slot0)
            issue_in(j0 + 2, slot0)
            wait_gather(slot1)
            compute_scatter(slot1)
            issue_in(jnp.minimum(j0 + 3, nch - 1), slot1)
            wait_in(slot0)
            issue_gather(slot0)
            return carry

        lax.fori_loop(0, (nch - 1) // 2, pairbody, 0)
        wait_gather(slot0)
        compute_scatter(slot0)
        wait_in(slot1)
        plsc.subcore_barrier()
        for kk in range(zr // CHUNK):
            off = s * zr + kk * CHUNK
            pltpu.sync_copy(s_sh.at[pl.ds(off, CHUNK)], grow0)
            pltpu.sync_copy(grow0, s_out.at[c, pl.ds(off, CHUNK)])

    return k(ewb, g, src3, dst3)


def _sc_degree(dst3, n_pad, d, e_total):
    nch = e_total // (NW * CHUNK)
    zr = n_pad // NS
    mesh = plsc.VectorSubcoreMesh(core_axis_name="c", subcore_axis_name="s")

    @functools.partial(
        pl.kernel,
        out_type=jax.ShapeDtypeStruct((NC, n_pad, d), jnp.float32),
        mesh=mesh,
        scratch_types=[
            pltpu.VMEM_SHARED((n_pad, d), jnp.float32),
            pltpu.VMEM((CHUNK,), jnp.int32),
            pltpu.VMEM((CHUNK, d), jnp.float32),
        ],
    )
    def k(dst_hbm, deg_out, deg_sh, dst_v, ones_v):
        c = lax.axis_index("c")
        s = lax.axis_index("s")
        wid = s * NC + c
        z16 = jnp.zeros((LANES,), jnp.float32)
        o16 = jnp.full((LANES,), 1.0, jnp.float32)

        def zrow(r, carry):
            for cc in range(d // LANES):
                ones_v[r, pl.ds(cc * LANES, LANES)] = z16
            return carry

        lax.fori_loop(0, CHUNK, zrow, 0)
        for kk in range(zr // CHUNK):
            pltpu.sync_copy(ones_v, deg_sh.at[pl.ds(s * zr + kk * CHUNK, CHUNK)])

        def orow(r, carry):
            for cc in range(d // LANES):
                ones_v[r, pl.ds(cc * LANES, LANES)] = o16
            return carry

        lax.fori_loop(0, CHUNK, orow, 0)
        plsc.subcore_barrier()

        def chunk_body(j, carry):
            pltpu.sync_copy(dst_hbm.at[wid, j], dst_v)
            pltpu.sync_copy(ones_v, deg_sh.at[dst_v], add=True)
            return carry

        lax.fori_loop(0, nch, chunk_body, 0)
        plsc.subcore_barrier()
        for kk in range(zr // CHUNK):
            off = s * zr + kk * CHUNK
            pltpu.sync_copy(deg_sh.at[pl.ds(off, CHUNK)], ones_v)
            pltpu.sync_copy(ones_v, deg_out.at[c, pl.ds(off, CHUNK)])

    return k(dst3)


def _final_mlp(s_parts, deg_parts, node, w1b, b1b, w2a, b2a, w2b, b2b, block_rows):
    n, d = node.shape

    def body(sp, dp, nd, w1, v1, w2, v2, w3, v3, o):
        seg = sp[0] + sp[1]
        deg = dp[0, :, 0:1] + dp[1, :, 0:1]
        agg = jnp.dot(seg, w1[...], preferred_element_type=jnp.float32) + deg * v1[...]
        h = agg + nd[...]
        hid = jnp.maximum(
            jnp.dot(h, w2[...], preferred_element_type=jnp.float32) + v2[...], 0.0)
        o[...] = jnp.dot(hid, w3[...], preferred_element_type=jnp.float32) + v3[...]

    wspec = pl.BlockSpec((d, d), lambda i: (0, 0))
    bspec = pl.BlockSpec((1, d), lambda i: (0, 0))
    return pl.pallas_call(
        body,
        grid=(n // block_rows,),
        in_specs=[
            pl.BlockSpec((NC, block_rows, d), lambda i: (0, i, 0)),
            pl.BlockSpec((NC, block_rows, d), lambda i: (0, i, 0)),
            pl.BlockSpec((block_rows, d), lambda i: (i, 0)),
            wspec, bspec, wspec, bspec, wspec, bspec,
        ],
        out_specs=pl.BlockSpec((block_rows, d), lambda i: (i, 0)),
        out_shape=jax.ShapeDtypeStruct((n, d), jnp.float32),
    )(s_parts, deg_parts, node, w1b, b1b, w2a, b2a, w2b, b2b)


def kernel(node_feats, edge_feats, edge_index, W1a, b1a, W1b, b1b, W2a, b2a, W2b, b2b):
    n, d = node_feats.shape
    e = edge_feats.shape[0]
    src = edge_index[0].astype(jnp.int32)
    dst = edge_index[1].astype(jnp.int32)
    nch = e // (NW * CHUNK)
    src3 = src.reshape(NW, nch, CHUNK)
    dst3 = dst.reshape(NW, nch, CHUNK)
    tile_rows = NS * CHUNK
    n_pad = ((n + tile_rows - 1) // tile_rows) * tile_rows

    g = _mm_bias(node_feats, W1a[:d], jnp.zeros((1, d), jnp.float32), 2000)
    ewb = _mm_bias(edge_feats, W1a[d:], b1a.reshape(1, d), 2560)
    s_parts = _sc_gather_relu_scatter(ewb, g, src3, dst3, n_pad)
    deg_parts = _sc_degree(dst3, n_pad, d, e)
    return _final_mlp(s_parts, deg_parts, node_feats,
                      W1b, b1b.reshape(1, d), W2a, b2a.reshape(1, d),
                      W2b, b2b.reshape(1, d), 2000)


# speedup vs baseline: 4.5564x; 1.2816x over previous
import functools

import jax
import jax.numpy as jnp
from jax import lax
from jax.experimental import pallas as pl
from jax.experimental.pallas import tpu as pltpu
from jax.experimental.pallas import tpu_sc as plsc

NC = 2
NS = 16
NW = NC * NS
LANES = 16
CHUNK = 80


def _mm_bias(x, w, b, block_rows):
    rows, k = x.shape
    dout = w.shape[1]

    def body(x_ref, w_ref, b_ref, o_ref):
        o_ref[...] = (
            jnp.dot(x_ref[...], w_ref[...], preferred_element_type=jnp.float32)
            + b_ref[...]
        )

    return pl.pallas_call(
        body,
        grid=(rows // block_rows,),
        in_specs=[
            pl.BlockSpec((block_rows, k), lambda i: (i, 0)),
            pl.BlockSpec((k, dout), lambda i: (0, 0)),
            pl.BlockSpec((1, dout), lambda i: (0, 0)),
        ],
        out_specs=pl.BlockSpec((block_rows, dout), lambda i: (i, 0)),
        out_shape=jax.ShapeDtypeStruct((rows, dout), jnp.float32),
    )(x, w, b)


def _sc_gather_relu_scatter(ewb, g, src3, dst3, n_pad):
    e_total, d = ewb.shape
    nch = e_total // (NW * CHUNK)
    zr = n_pad // NS
    mesh = plsc.VectorSubcoreMesh(core_axis_name="c", subcore_axis_name="s")

    assert nch % 2 == 1 and nch >= 3

    @functools.partial(
        pl.kernel,
        out_type=jax.ShapeDtypeStruct((NC, n_pad, d), jnp.float32),
        mesh=mesh,
        scratch_types=[
            pltpu.VMEM_SHARED((n_pad, d), jnp.float32),
            pltpu.VMEM((CHUNK,), jnp.int32),
            pltpu.VMEM((CHUNK,), jnp.int32),
            pltpu.VMEM((CHUNK,), jnp.int32),
            pltpu.VMEM((CHUNK,), jnp.int32),
            pltpu.VMEM((CHUNK, d), jnp.float32),
            pltpu.VMEM((CHUNK, d), jnp.float32),
            pltpu.VMEM((CHUNK, d), jnp.float32),
            pltpu.VMEM((CHUNK, d), jnp.float32),
            pltpu.SemaphoreType.DMA,
            pltpu.SemaphoreType.DMA,
            pltpu.SemaphoreType.DMA,
            pltpu.SemaphoreType.DMA,
        ],
    )
    def k(ewb_hbm, g_hbm, src_hbm, dst_hbm, s_out,
          s_sh, src0, dst0, src1, dst1,
          ewb0, ewb1, grow0, grow1, sin0, sin1, gs0, gs1):
        c = lax.axis_index("c")
        s = lax.axis_index("s")
        wid = s * NC + c
        z16 = jnp.zeros((LANES,), jnp.float32)
        slot0 = (src0, dst0, ewb0, grow0, sin0, gs0)
        slot1 = (src1, dst1, ewb1, grow1, sin1, gs1)

        def issue_in(j, sl):
            src_b, dst_b, ewb_b, _, sem, _ = sl
            pltpu.async_copy(src_hbm.at[wid, j], src_b, sem)
            pltpu.async_copy(dst_hbm.at[wid, j], dst_b, sem)
            base = (wid * nch + j) * CHUNK
            pltpu.async_copy(ewb_hbm.at[pl.ds(base, CHUNK)], ewb_b, sem)

        def wait_in(sl):
            src_b, dst_b, ewb_b, _, sem, _ = sl
            pltpu.make_async_copy(src_hbm.at[wid, 0], src_b, sem).wait()
            pltpu.make_async_copy(dst_hbm.at[wid, 0], dst_b, sem).wait()
            pltpu.make_async_copy(ewb_hbm.at[pl.ds(0, CHUNK)], ewb_b, sem).wait()

        def issue_gather(sl):
            src_b, _, _, grow_b, _, gsem = sl
            pltpu.async_copy(g_hbm.at[src_b], grow_b, gsem)

        def wait_gather(sl):
            src_b, _, _, grow_b, _, gsem = sl
            pltpu.make_async_copy(g_hbm.at[src_b], grow_b, gsem).wait()

        def compute_scatter(sl):
            _, dst_b, ewb_b, grow_b, _, _ = sl

            def vrow(r, rcarry):
                for cc in range(d // LANES):
                    vs = pl.ds(cc * LANES, LANES)
                    grow_b[r, vs] = jnp.maximum(grow_b[r, vs] + ewb_b[r, vs], 0.0)
                return rcarry

            lax.fori_loop(0, CHUNK, vrow, 0)
            pltpu.sync_copy(grow_b, s_sh.at[dst_b], add=True)

        def zrow(r, carry):
            for cc in range(d // LANES):
                grow0[r, pl.ds(cc * LANES, LANES)] = z16
            return carry

        lax.fori_loop(0, CHUNK, zrow, 0)
        for kk in range(zr // CHUNK):
            pltpu.sync_copy(grow0, s_sh.at[pl.ds(s * zr + kk * CHUNK, CHUNK)])

        plsc.subcore_barrier()

        issue_in(0, slot0)
        issue_in(1, slot1)
        wait_in(slot0)
        issue_gather(slot0)

        def pairbody(kk, carry):
            j0 = 2 * kk
            wait_in(slot1)
            issue_gather(slot1)
            wait_gather(slot0)
            compute_scatter(slot0)
            issue_in(j0 + 2, slot0)
            wait_gather(slot1)
            compute_scatter(slot1)
            issue_in(jnp.minimum(j0 + 3, nch - 1), slot1)
            wait_in(slot0)
            issue_gather(slot0)
            return carry

        lax.fori_loop(0, (nch - 1) // 2, pairbody, 0)
        wait_gather(slot0)
        compute_scatter(slot0)
        wait_in(slot1)
        plsc.subcore_barrier()
        for kk in range(zr // CHUNK):
            off = s * zr + kk * CHUNK
            pltpu.sync_copy(s_sh.at[pl.ds(off, CHUNK)], grow0)
            pltpu.sync_copy(grow0, s_out.at[c, pl.ds(off, CHUNK)])

    return k(ewb, g, src3, dst3)


def _sc_degree(dst3, n_pad, e_total):
    nch = e_total // (NW * CHUNK)
    slots = 5
    assert nch % slots == 0
    groups = nch // slots
    dzr = n_pad // NS
    mesh = plsc.VectorSubcoreMesh(core_axis_name="c", subcore_axis_name="s")

    @functools.partial(
        pl.kernel,
        out_type=jax.ShapeDtypeStruct((NC, n_pad), jnp.float32),
        mesh=mesh,
        scratch_types=[
            pltpu.VMEM_SHARED((n_pad,), jnp.float32),
            pltpu.VMEM((n_pad,), jnp.float32),
            pltpu.VMEM((CHUNK,), jnp.float32),
            [pltpu.VMEM((CHUNK,), jnp.int32) for _ in range(slots)],
            [pltpu.SemaphoreType.DMA for _ in range(slots)],
            pltpu.SemaphoreType.DMA,
        ],
    )
    def k(dst_hbm, deg_out, deg_sh, deg_t, ones_c, idx_b, isem, ssem):
        c = lax.axis_index("c")
        s = lax.axis_index("s")
        wid = s * NC + c
        z16 = jnp.zeros((LANES,), jnp.float32)
        o16 = jnp.full((LANES,), 1.0, jnp.float32)

        def zdeg(i, carry):
            deg_t[pl.ds(i * LANES, LANES)] = z16
            return carry

        lax.fori_loop(0, n_pad // LANES, zdeg, 0)

        @pl.when(s == 0)
        def _():
            pltpu.sync_copy(deg_t, deg_sh)

        def orow(i, carry):
            ones_c[pl.ds(i * LANES, LANES)] = o16
            return carry

        lax.fori_loop(0, CHUNK // LANES, orow, 0)
        plsc.subcore_barrier()

        for b in range(slots):
            pltpu.async_copy(dst_hbm.at[wid, b], idx_b[b], isem[b])

        def gbody(gg, carry):
            for b in range(slots):
                pltpu.make_async_copy(dst_hbm.at[wid, 0], idx_b[b], isem[b]).wait()
                pltpu.async_copy(ones_c, deg_sh.at[idx_b[b]], ssem, add=True)
            for b in range(slots):
                pltpu.make_async_copy(ones_c, deg_sh.at[idx_b[b]], ssem).wait()
            jn = (gg + 1) * slots
            for b in range(slots):
                pltpu.async_copy(
                    dst_hbm.at[wid, jnp.minimum(jn + b, nch - 1)], idx_b[b], isem[b])
            return carry

        lax.fori_loop(0, groups, gbody, 0)
        for b in range(slots):
            pltpu.make_async_copy(dst_hbm.at[wid, 0], idx_b[b], isem[b]).wait()
        plsc.subcore_barrier()
        pltpu.sync_copy(deg_sh.at[pl.ds(s * dzr, dzr)], deg_t.at[pl.ds(0, dzr)])
        pltpu.sync_copy(deg_t.at[pl.ds(0, dzr)], deg_out.at[c, pl.ds(s * dzr, dzr)])

    return k(dst3)


def _sum_partials(dp):
    nparts, npd = dp.shape
    blk = 1024

    def body(d_ref, o_ref):
        o_ref[...] = jnp.sum(d_ref[...], axis=0)[:, None]

    return pl.pallas_call(
        body,
        grid=(npd // blk,),
        in_specs=[pl.BlockSpec((nparts, blk), lambda i: (0, i))],
        out_specs=pl.BlockSpec((blk, 1), lambda i: (i, 0)),
        out_shape=jax.ShapeDtypeStruct((npd, 1), jnp.float32),
    )(dp)


def _final_mlp(s_parts, deg, node, w1b, b1b, w2a, b2a, w2b, b2b, block_rows):
    n, d = node.shape

    def body(sp, dp, nd, w1, v1, w2, v2, w3, v3, o):
        seg = sp[0] + sp[1]
        agg = jnp.dot(seg, w1[...], preferred_element_type=jnp.float32) + dp[...] * v1[...]
        h = agg + nd[...]
        hid = jnp.maximum(
            jnp.dot(h, w2[...], preferred_element_type=jnp.float32) + v2[...], 0.0)
        o[...] = jnp.dot(hid, w3[...], preferred_element_type=jnp.float32) + v3[...]

    wspec = pl.BlockSpec((d, d), lambda i: (0, 0))
    bspec = pl.BlockSpec((1, d), lambda i: (0, 0))
    return pl.pallas_call(
        body,
        grid=(n // block_rows,),
        in_specs=[
            pl.BlockSpec((NC, block_rows, d), lambda i: (0, i, 0)),
            pl.BlockSpec((block_rows, 1), lambda i: (i, 0)),
            pl.BlockSpec((block_rows, d), lambda i: (i, 0)),
            wspec, bspec, wspec, bspec, wspec, bspec,
        ],
        out_specs=pl.BlockSpec((block_rows, d), lambda i: (i, 0)),
        out_shape=jax.ShapeDtypeStruct((n, d), jnp.float32),
    )(s_parts, deg, node, w1b, b1b, w2a, b2a, w2b, b2b)


def kernel(node_feats, edge_feats, edge_index, W1a, b1a, W1b, b1b, W2a, b2a, W2b, b2b):
    n, d = node_feats.shape
    e = edge_feats.shape[0]
    src = edge_index[0].astype(jnp.int32)
    dst = edge_index[1].astype(jnp.int32)
    nch = e // (NW * CHUNK)
    src3 = src.reshape(NW, nch, CHUNK)
    dst3 = dst.reshape(NW, nch, CHUNK)
    tile_rows = NS * CHUNK
    n_pad = ((n + tile_rows - 1) // tile_rows) * tile_rows

    g = _mm_bias(node_feats, W1a[:d], jnp.zeros((1, d), jnp.float32), 2000)
    ewb = _mm_bias(edge_feats, W1a[d:], b1a.reshape(1, d), 2560)
    s_parts = _sc_gather_relu_scatter(ewb, g, src3, dst3, n_pad)
    deg_parts = _sc_degree(dst3, n_pad, e)
    deg = _sum_partials(deg_parts)
    return _final_mlp(s_parts, deg, node_feats,
                      W1b, b1b.reshape(1, d), W2a, b2a.reshape(1, d),
                      W2b, b2b.reshape(1, d), 2000)
